# baseline (speedup 1.0000x reference)
import functools

import jax
import jax.numpy as jnp
import numpy as np
from jax.experimental import pallas as pl


def _body(ncat, x_ref, m_ref, cf_ref, nap_ref, o_ref):
    xb = x_ref[...]
    nan = jnp.isnan(xb)
    xc = jnp.where(nan, 0.0, xb)
    idx = xc[:, :ncat].astype(jnp.int32)
    parts = [xc]
    for v in range(7):
        parts.append(jnp.where(idx == v, 1.0, 0.0))
    xin = jnp.concatenate(parts, axis=1).astype(jnp.bfloat16)
    y = jax.lax.dot_general(
        xin, m_ref[...], (((1,), (0,)), ((), ())),
        preferred_element_type=jnp.float32)
    y = y + cf_ref[...]
    H = y.shape[1] // xb.shape[1]
    nanf = jnp.where(nan, 1.0, 0.0)
    nan_g = jnp.concatenate([nanf] * H, axis=1)
    y = jnp.where(nan_g != 0, nap_ref[...], y)
    g = jax.lax.erf(y * 0.7071067811865476)
    a = 0.5 * y
    o_ref[...] = a + a * g


def kernel(x, cat_tables, lin_w, lin_b, na_emb, pos_table):
    B, T, D = x.shape
    NCAT, V, H = cat_tables.shape
    BT = B * T
    DH = D * H
    xf = x.reshape(BT, D)

    kk = np.arange(DH)
    hh = kk // D
    dd = kk % D
    ar = np.arange(DH)

    zc = jnp.zeros((NCAT, H), dtype=lin_w.dtype)
    lin_w_pad = jnp.concatenate([zc, lin_w], axis=0)
    lin_b_pad = jnp.concatenate([zc, lin_b], axis=0)

    K = D + 7 * NCAT
    m = jnp.zeros((K, DH), dtype=jnp.float32)
    m = m.at[dd, ar].set(lin_w_pad[dd, hh])
    ddc = np.minimum(dd, NCAT - 1)
    for v in range(V):
        vals = jnp.where(dd < NCAT, cat_tables[ddc, v, hh], 0.0)
        m = m.at[D + v * NCAT + ddc, ar].add(vals)
    m = m.astype(jnp.bfloat16)

    pos_flat = pos_table.reshape(DH)
    cf = (pos_flat + lin_b_pad[dd, hh]).reshape(1, DH)
    nap = (pos_flat + na_emb[0][hh]).reshape(1, DH)

    R = 256
    while BT % R:
        R //= 2

    out = pl.pallas_call(
        functools.partial(_body, NCAT),
        grid=(BT // R,),
        in_specs=[
            pl.BlockSpec((R, D), lambda i: (i, 0)),
            pl.BlockSpec((K, DH), lambda i: (0, 0)),
            pl.BlockSpec((1, DH), lambda i: (0, 0)),
            pl.BlockSpec((1, DH), lambda i: (0, 0)),
        ],
        out_specs=pl.BlockSpec((R, DH), lambda i: (i, 0)),
        out_shape=jax.ShapeDtypeStruct((BT, DH), jnp.float32),
    )(xf, m, cf, nap)
    return out.reshape(B, T, DH)


# speedup vs baseline: 2.2001x; 2.2001x over previous
import functools

import jax
import jax.numpy as jnp
import numpy as np
from jax.experimental import pallas as pl


def _body(ncat, x_ref, m_ref, nap_ref, o_ref):
    xb = x_ref[...]
    nan = jnp.isnan(xb)
    xc = jnp.where(nan, 0.0, xb)
    idx = xc[:, :ncat].astype(jnp.int32)
    parts = [xc]
    for v in range(7):
        parts.append(jnp.where(idx == v, 1.0, 0.0))
    parts.append(jnp.ones((xb.shape[0], 2), dtype=xb.dtype))
    xin = jnp.concatenate(parts, axis=1).astype(jnp.bfloat16)
    y = jax.lax.dot_general(
        xin, m_ref[...], (((1,), (0,)), ((), ())),
        preferred_element_type=jnp.float32)
    H = y.shape[1] // xb.shape[1]
    nanf = jnp.where(nan, 1.0, 0.0)
    nan_g = jnp.concatenate([nanf] * H, axis=1)
    y = jnp.where(nan_g != 0, nap_ref[...], y)
    g = jax.lax.erf(y * 0.7071067811865476)
    o_ref[...] = y * (0.5 + 0.5 * g)


def kernel(x, cat_tables, lin_w, lin_b, na_emb, pos_table):
    B, T, D = x.shape
    NCAT, V, H = cat_tables.shape
    BT = B * T
    DH = D * H
    xf = x.reshape(BT, D)

    kk = np.arange(DH)
    hh = kk // D
    dd = kk % D

    onehot_h = (np.arange(H)[:, None] == hh[None, :]).astype(np.float32)
    eq_w = np.arange(D)[:, None] == dd[None, :]
    eq_c = (dd[None, :] == (np.arange(7 * NCAT) % NCAT)[:, None]) \
        & (dd[None, :] < NCAT)

    zc = jnp.zeros((NCAT, H), dtype=lin_w.dtype)
    lin_w_pad = jnp.concatenate([zc, lin_w], axis=0)
    lin_b_pad = jnp.concatenate([zc, lin_b], axis=0)

    w_rows = jnp.where(eq_w, lin_w_pad @ onehot_h, 0.0)
    ctv = cat_tables.transpose(1, 0, 2).reshape(V * NCAT, H)
    c_rows = jnp.where(eq_c[: V * NCAT], ctv @ onehot_h, 0.0)
    if V < 7:
        c_rows = jnp.concatenate(
            [c_rows, jnp.zeros(((7 - V) * NCAT, DH), c_rows.dtype)], axis=0)

    pos_flat = pos_table.reshape(DH)
    cf = pos_flat + jnp.sum(jnp.where(eq_w, lin_b_pad @ onehot_h, 0.0), axis=0)
    cf_hi = cf.astype(jnp.bfloat16).astype(jnp.float32)
    cf_lo = cf - cf_hi
    m = jnp.concatenate(
        [w_rows, c_rows, cf_hi[None, :], cf_lo[None, :]],
        axis=0).astype(jnp.bfloat16)
    K = D + 7 * NCAT + 2

    nap = (pos_flat + (na_emb @ jnp.asarray(onehot_h))[0]).reshape(1, DH)

    R = 512
    while BT % R:
        R //= 2

    out = pl.pallas_call(
        functools.partial(_body, NCAT),
        grid=(BT // R,),
        in_specs=[
            pl.BlockSpec((R, D), lambda i: (i, 0)),
            pl.BlockSpec((K, DH), lambda i: (0, 0)),
            pl.BlockSpec((1, DH), lambda i: (0, 0)),
        ],
        out_specs=pl.BlockSpec((R, DH), lambda i: (i, 0)),
        out_shape=jax.ShapeDtypeStruct((BT, DH), jnp.float32),
    )(xf, m, nap)
    return out.reshape(B, T, DH)
